# baseline (speedup 1.0000x reference)
import functools

import jax
import jax.numpy as jnp
from jax import lax
from jax.experimental import pallas as pl
from jax.experimental.pallas import tpu as pltpu
from jax.experimental.pallas import tpu_sc as plsc

F32 = jnp.float32
DN_NT = (((1,), (1,)), ((), ()))


def _amax_body(t_ref, am_ref):
    t = t_ref[...]
    B, L = t.shape
    mx = jnp.max(t, axis=1, keepdims=True)
    col = lax.broadcasted_iota(jnp.int32, (B, L), 1)
    cand = jnp.where(t == mx, col, L)
    am = jnp.min(cand, axis=1, keepdims=True)
    am_ref[...] = jnp.broadcast_to(am, (B, 8))


def _select_body(am_ref, t_ref, a_ref, gid_ref, rows_ref, sem, *, B, L, KP):
    def issue(i, c):
        amb = am_ref[i, 0]
        pltpu.make_async_copy(a_ref.at[i, amb], rows_ref.at[i], sem).start()
        return c

    lax.fori_loop(0, B, issue, 0)

    def drain(i, c):
        pltpu.make_async_copy(a_ref.at[0, 0], rows_ref.at[0], sem).wait()
        return c

    lax.fori_loop(0, B, drain, 0)

    J = lax.broadcasted_iota(jnp.int32, (L, L), 1)
    I = lax.broadcasted_iota(jnp.int32, (L, L), 0)
    R = lax.broadcasted_iota(jnp.int32, (L, KP), 1)
    Irow = lax.broadcasted_iota(jnp.int32, (L, KP), 0).astype(F32)

    def body(b, c):
        amb = am_ref[b, 0]
        t = t_ref[pl.ds(b, 1), :]
        a = rows_ref[pl.ds(b, 1), :]
        col1 = lax.broadcasted_iota(jnp.int32, (1, L), 1)
        a = jnp.where(col1 == amb, jnp.float32(-1.0), a)
        a = jnp.where(t != 0, a, jnp.float32(0.0))

        A = jnp.broadcast_to(a, (L, L))
        AT = lax.transpose(A, (1, 0))
        beat = (A > AT) | ((A == AT) & (J < I))
        rank = jnp.sum(beat.astype(F32), axis=1, keepdims=True)

        ranki = rank.astype(jnp.int32)
        O = (ranki == R).astype(F32)
        idxf = jnp.sum(O * Irow, axis=0, keepdims=True)
        gid_ref[pl.ds(b, 1), :] = idxf.astype(jnp.int32) + b * L
        return c

    lax.fori_loop(0, B, body, 0)


def _sc_gather_body(tab_ref, idx_ref, out_ref, idx_v, rows_v, sem,
                    *, per_w, ch):
    wid = lax.axis_index("s") * 2 + lax.axis_index("c")
    for c in range(per_w // ch):
        base = wid * per_w + c * ch
        pltpu.sync_copy(idx_ref.at[pl.ds(base, ch)], idx_v)
        pltpu.async_copy(tab_ref.at[idx_v], rows_v, sem).wait()
        pltpu.sync_copy(rows_v, out_ref.at[pl.ds(base, ch)])


def _stats_body(x_ref, w1_ref, b1_ref, st_ref, *, K, KP):
    i = pl.program_id(0)

    @pl.when(i == 0)
    def _():
        st_ref[...] = jnp.zeros_like(st_ref)

    x = x_ref[...]
    nrm = jnp.sqrt(jnp.sum(x * x, axis=1, keepdims=True)) + 1e-8
    xn = x / nrm
    h = lax.dot_general(xn, w1_ref[...], DN_NT, preferred_element_type=F32)
    h = h + b1_ref[...]
    rows = lax.broadcasted_iota(jnp.int32, (h.shape[0], 1), 0)
    real = (rows % KP) < K
    hm = jnp.where(real, h, 0.0)
    st_ref[0:1, :] += jnp.sum(hm, axis=0, keepdims=True)
    st_ref[1:2, :] += jnp.sum(hm * hm, axis=0, keepdims=True)


def _final_body(x_ref, w1_ref, b1_ref, st_ref, g_ref, bb_ref, w2_ref,
                b2_ref, lw_ref, lb_ref, out_ref, *, n_rows, K, KP):
    x = x_ref[...]
    nrm = jnp.sqrt(jnp.sum(x * x, axis=1, keepdims=True)) + 1e-8
    xn = x / nrm
    h = lax.dot_general(xn, w1_ref[...], DN_NT, preferred_element_type=F32)
    h = h + b1_ref[...]
    inv_n = jnp.float32(1.0 / n_rows)
    mean = st_ref[0:1, :] * inv_n
    var = st_ref[1:2, :] * inv_n - mean * mean
    hn = (h - mean) / jnp.sqrt(var + 1e-5) * g_ref[...] + bb_ref[...]
    hr = jnp.maximum(hn, 0.0)
    out = lax.dot_general(hr, w2_ref[...], DN_NT, preferred_element_type=F32)
    out = out + b2_ref[...]
    c = xn * jnp.float32(8193.0)
    hi = c - (c - xn)
    mg = jnp.float32(0.75)
    sub = (xn + mg) - mg
    x16 = jnp.where(jnp.abs(xn) < jnp.float32(6.103515625e-05), sub, hi)
    cap = lax.dot_general(x16, lw_ref[...], DN_NT, preferred_element_type=F32)
    res = out + cap + lb_ref[...]
    for j in range(8):
        out_ref[j] = res[j * KP:j * KP + K, :]


def kernel(features, text, atten, linear_W, linear_b, mlp_W1, mlp_b1,
           bn_gamma, bn_beta, mlp_W2, mlp_b2):
    B, L, D = features.shape
    DE = linear_W.shape[0]
    H = mlp_W1.shape[0]
    K = (atten.shape[1] - 2) * 3 // 10
    KP = 160
    N = B * K
    TOT = B * KP

    am8 = pl.pallas_call(
        _amax_body,
        out_shape=jax.ShapeDtypeStruct((B, 8), jnp.int32),
        in_specs=[pl.BlockSpec((B, L), lambda: (0, 0))],
        out_specs=pl.BlockSpec((B, 8), lambda: (0, 0)),
    )(text)

    gid = pl.pallas_call(
        functools.partial(_select_body, B=B, L=L, KP=KP),
        in_specs=[
            pl.BlockSpec(memory_space=pltpu.MemorySpace.SMEM),
            pl.BlockSpec((B, L), lambda: (0, 0)),
            pl.BlockSpec(memory_space=pltpu.MemorySpace.HBM),
        ],
        out_specs=pl.BlockSpec((B, KP), lambda: (0, 0)),
        out_shape=jax.ShapeDtypeStruct((B, KP), jnp.int32),
        scratch_shapes=[
            pltpu.VMEM((B, L), F32),
            pltpu.SemaphoreType.DMA,
        ],
    )(am8, text, atten)

    NW = 32
    per_w = TOT // NW
    CH = 128
    mesh = plsc.VectorSubcoreMesh(core_axis_name="c", subcore_axis_name="s")
    xg = pl.kernel(
        functools.partial(_sc_gather_body, per_w=per_w, ch=CH),
        mesh=mesh,
        out_type=jax.ShapeDtypeStruct((TOT, D), F32),
        scratch_types=[
            pltpu.VMEM((CH,), jnp.int32),
            pltpu.VMEM((CH, D), F32),
            pltpu.SemaphoreType.DMA,
        ],
        compiler_params=pltpu.CompilerParams(use_tc_tiling_on_sc=True),
    )(features.reshape(B * L, D), gid.reshape(TOT))

    RB = 8 * KP
    n_steps = TOT // RB
    stats = pl.pallas_call(
        functools.partial(_stats_body, K=K, KP=KP),
        grid=(n_steps,),
        in_specs=[
            pl.BlockSpec((RB, D), lambda i: (i, 0)),
            pl.BlockSpec((H, D), lambda i: (0, 0)),
            pl.BlockSpec((1, H), lambda i: (0, 0)),
        ],
        out_specs=pl.BlockSpec((8, H), lambda i: (0, 0)),
        out_shape=jax.ShapeDtypeStruct((8, H), F32),
    )(xg, mlp_W1, mlp_b1.reshape(1, H))

    res = pl.pallas_call(
        functools.partial(_final_body, n_rows=N, K=K, KP=KP),
        grid=(n_steps,),
        in_specs=[
            pl.BlockSpec((RB, D), lambda i: (i, 0)),
            pl.BlockSpec((H, D), lambda i: (0, 0)),
            pl.BlockSpec((1, H), lambda i: (0, 0)),
            pl.BlockSpec((8, H), lambda i: (0, 0)),
            pl.BlockSpec((1, H), lambda i: (0, 0)),
            pl.BlockSpec((1, H), lambda i: (0, 0)),
            pl.BlockSpec((DE, H), lambda i: (0, 0)),
            pl.BlockSpec((1, DE), lambda i: (0, 0)),
            pl.BlockSpec((DE, D), lambda i: (0, 0)),
            pl.BlockSpec((1, DE), lambda i: (0, 0)),
        ],
        out_specs=pl.BlockSpec((8, K, DE), lambda i: (i, 0, 0)),
        out_shape=jax.ShapeDtypeStruct((B, K, DE), F32),
    )(xg, mlp_W1, mlp_b1.reshape(1, H), stats, bn_gamma.reshape(1, H),
      bn_beta.reshape(1, H), mlp_W2, mlp_b2.reshape(1, DE), linear_W,
      linear_b.reshape(1, DE))

    return res


# speedup vs baseline: 1.3817x; 1.3817x over previous
import functools

import jax
import jax.numpy as jnp
from jax import lax
from jax.experimental import pallas as pl
from jax.experimental.pallas import tpu as pltpu
from jax.experimental.pallas import tpu_sc as plsc

F32 = jnp.float32
DN_NT = (((1,), (1,)), ((), ()))


def _amax_body(t_ref, am_ref):
    t = t_ref[...]
    B, L = t.shape
    mx = jnp.max(t, axis=1, keepdims=True)
    col = lax.broadcasted_iota(jnp.int32, (B, L), 1)
    cand = jnp.where(t == mx, col, L)
    am = jnp.min(cand, axis=1, keepdims=True)
    am_ref[...] = jnp.broadcast_to(am, (B, 8))


def _select_body(am_ref, t_ref, a_ref, gid_ref, rows_ref, gid_scr, sem,
                 *, B, L, KP):
    def issue(i, c):
        amb = am_ref[i, 0]
        pltpu.make_async_copy(a_ref.at[i, amb], rows_ref.at[i], sem).start()
        return c

    lax.fori_loop(0, B, issue, 0)

    def drain(i, c):
        pltpu.make_async_copy(a_ref.at[0, 0], rows_ref.at[0], sem).wait()
        return c

    lax.fori_loop(0, B, drain, 0)

    J = lax.broadcasted_iota(jnp.int32, (L, L), 1)
    I = lax.broadcasted_iota(jnp.int32, (L, L), 0)
    R = lax.broadcasted_iota(jnp.int32, (L, KP), 1)
    Irow = lax.broadcasted_iota(jnp.int32, (L, KP), 0).astype(F32)

    def body(b, c):
        amb = am_ref[b, 0]
        t = t_ref[pl.ds(b, 1), :]
        a = rows_ref[pl.ds(b, 1), :]
        col1 = lax.broadcasted_iota(jnp.int32, (1, L), 1)
        a = jnp.where(col1 == amb, jnp.float32(-1.0), a)
        a = jnp.where(t != 0, a, jnp.float32(0.0))

        A = jnp.broadcast_to(a, (L, L))
        AT = lax.transpose(A, (1, 0))
        beat = (A > AT) | ((A == AT) & (J < I))
        rank = jnp.sum(beat.astype(F32), axis=1, keepdims=True)

        ranki = rank.astype(jnp.int32)
        O = (ranki == R).astype(F32)
        idxf = jnp.sum(O * Irow, axis=0, keepdims=True)
        gid_scr[pl.ds(b, 1), :] = idxf.astype(jnp.int32) + b * L
        return c

    lax.fori_loop(0, B, body, 0)
    gid_ref[...] = lax.transpose(gid_scr[...], (1, 0))


def _sc_gather_body(tab_ref, idx_ref, out_ref, idx_v, rows_v, sem,
                    *, per_w, ch):
    wid = lax.axis_index("s") * 2 + lax.axis_index("c")
    for c in range(per_w // ch):
        base = wid * per_w + c * ch
        pltpu.sync_copy(idx_ref.at[pl.ds(base, ch)], idx_v)
        pltpu.async_copy(tab_ref.at[idx_v], rows_v, sem).wait()
        pltpu.sync_copy(rows_v, out_ref.at[pl.ds(base, ch)])


def _stats_body(x_ref, w1_ref, b1_ref, st_ref):
    i = pl.program_id(0)

    @pl.when(i == 0)
    def _():
        st_ref[...] = jnp.zeros_like(st_ref)

    x = x_ref[...]
    nrm = jnp.sqrt(jnp.sum(x * x, axis=1, keepdims=True)) + 1e-8
    xn = x / nrm
    h = lax.dot_general(xn, w1_ref[...], DN_NT, preferred_element_type=F32)
    h = h + b1_ref[...]
    st_ref[0:1, :] += jnp.sum(h, axis=0, keepdims=True)
    st_ref[1:2, :] += jnp.sum(h * h, axis=0, keepdims=True)


def _final_body(x_ref, w1_ref, b1_ref, st_ref, g_ref, bb_ref, w2_ref,
                b2_ref, lw_ref, lb_ref, out_ref, *, n_rows):
    x = x_ref[...]
    nrm = jnp.sqrt(jnp.sum(x * x, axis=1, keepdims=True)) + 1e-8
    xn = x / nrm
    h = lax.dot_general(xn, w1_ref[...], DN_NT, preferred_element_type=F32)
    h = h + b1_ref[...]
    inv_n = jnp.float32(1.0 / n_rows)
    mean = st_ref[0:1, :] * inv_n
    var = st_ref[1:2, :] * inv_n - mean * mean
    hn = (h - mean) / jnp.sqrt(var + 1e-5) * g_ref[...] + bb_ref[...]
    hr = jnp.maximum(hn, 0.0)
    out = lax.dot_general(hr, w2_ref[...], DN_NT, preferred_element_type=F32)
    out = out + b2_ref[...]
    c = xn * jnp.float32(8193.0)
    hi = c - (c - xn)
    mg = jnp.float32(0.75)
    sub = (xn + mg) - mg
    x16 = jnp.where(jnp.abs(xn) < jnp.float32(6.103515625e-05), sub, hi)
    cap = lax.dot_general(x16, lw_ref[...], DN_NT, preferred_element_type=F32)
    out_ref[...] = out + cap + lb_ref[...]


def kernel(features, text, atten, linear_W, linear_b, mlp_W1, mlp_b1,
           bn_gamma, bn_beta, mlp_W2, mlp_b2):
    B, L, D = features.shape
    DE = linear_W.shape[0]
    H = mlp_W1.shape[0]
    K = (atten.shape[1] - 2) * 3 // 10
    KP = 160
    N = B * K
    TOT = B * KP

    am8 = pl.pallas_call(
        _amax_body,
        out_shape=jax.ShapeDtypeStruct((B, 8), jnp.int32),
        in_specs=[pl.BlockSpec((B, L), lambda: (0, 0))],
        out_specs=pl.BlockSpec((B, 8), lambda: (0, 0)),
    )(text)

    gid = pl.pallas_call(
        functools.partial(_select_body, B=B, L=L, KP=KP),
        in_specs=[
            pl.BlockSpec(memory_space=pltpu.MemorySpace.SMEM),
            pl.BlockSpec((B, L), lambda: (0, 0)),
            pl.BlockSpec(memory_space=pltpu.MemorySpace.HBM),
        ],
        out_specs=pl.BlockSpec((KP, B), lambda: (0, 0)),
        out_shape=jax.ShapeDtypeStruct((KP, B), jnp.int32),
        scratch_shapes=[
            pltpu.VMEM((B, L), F32),
            pltpu.VMEM((B, KP), jnp.int32),
            pltpu.SemaphoreType.DMA,
        ],
    )(am8, text, atten)

    NW = 32
    per_w = TOT // NW
    CH = 128
    mesh = plsc.VectorSubcoreMesh(core_axis_name="c", subcore_axis_name="s")
    xg = pl.kernel(
        functools.partial(_sc_gather_body, per_w=per_w, ch=CH),
        mesh=mesh,
        out_type=jax.ShapeDtypeStruct((TOT, D), F32),
        scratch_types=[
            pltpu.VMEM((CH,), jnp.int32),
            pltpu.VMEM((CH, D), F32),
            pltpu.SemaphoreType.DMA,
        ],
        compiler_params=pltpu.CompilerParams(use_tc_tiling_on_sc=True),
    )(features.reshape(B * L, D), gid.reshape(TOT))

    RB = 8 * K
    n_steps = N // RB
    stats = pl.pallas_call(
        _stats_body,
        grid=(n_steps,),
        in_specs=[
            pl.BlockSpec((RB, D), lambda i: (i, 0)),
            pl.BlockSpec((H, D), lambda i: (0, 0)),
            pl.BlockSpec((1, H), lambda i: (0, 0)),
        ],
        out_specs=pl.BlockSpec((8, H), lambda i: (0, 0)),
        out_shape=jax.ShapeDtypeStruct((8, H), F32),
    )(xg, mlp_W1, mlp_b1.reshape(1, H))

    res = pl.pallas_call(
        functools.partial(_final_body, n_rows=N),
        grid=(n_steps,),
        in_specs=[
            pl.BlockSpec((RB, D), lambda i: (i, 0)),
            pl.BlockSpec((H, D), lambda i: (0, 0)),
            pl.BlockSpec((1, H), lambda i: (0, 0)),
            pl.BlockSpec((8, H), lambda i: (0, 0)),
            pl.BlockSpec((1, H), lambda i: (0, 0)),
            pl.BlockSpec((1, H), lambda i: (0, 0)),
            pl.BlockSpec((DE, H), lambda i: (0, 0)),
            pl.BlockSpec((1, DE), lambda i: (0, 0)),
            pl.BlockSpec((DE, D), lambda i: (0, 0)),
            pl.BlockSpec((1, DE), lambda i: (0, 0)),
        ],
        out_specs=pl.BlockSpec((RB, DE), lambda i: (i, 0)),
        out_shape=jax.ShapeDtypeStruct((N, DE), F32),
    )(xg, mlp_W1, mlp_b1.reshape(1, H), stats, bn_gamma.reshape(1, H),
      bn_beta.reshape(1, H), mlp_W2, mlp_b2.reshape(1, DE), linear_W,
      linear_b.reshape(1, DE))

    return res.reshape(K, B, DE).swapaxes(0, 1)
